# Initial kernel scaffold; baseline (speedup 1.0000x reference)
import jax
import jax.numpy as jnp
from jax.experimental import pallas as pl


def kernel(x, edge_index, edge_attr, Wl1, Wr1, We1, att1, b1, Wl2, Wr2, We2, att2, b2, Wl3, Wr3, We3, att3, b3, Wlin, blin, Wlin1, blin1):
    raise NotImplementedError("write your pallas kernel here")


# speedup vs baseline: 1.6350x; 1.6350x over previous
import functools

import jax
import jax.numpy as jnp
from jax import lax
from jax.experimental import pallas as pl
from jax.experimental.pallas import tpu as pltpu

N = 50000
E = 800000
NB = 256
NP = 50176


def _leaky(x):
    return jnp.maximum(x, 0.2 * x)


def _mlp_body(h_ref, wlin_ref, blin_ref, wlin1_ref, blin1_ref, o_ref):
    t = jnp.maximum(h_ref[...] @ wlin_ref[...] + blin_ref[...], 0.0)
    o_ref[...] = t @ wlin1_ref[...] + blin1_ref[...]


def _mlp_tail(h, Wlin, blin, Wlin1, blin1):
    w1p = jnp.zeros((32, 128), jnp.float32).at[:, :4].set(Wlin1)
    b1p = jnp.zeros((128,), jnp.float32).at[:4].set(blin1)
    grid = NP // NB
    return pl.pallas_call(
        _mlp_body,
        grid=(grid,),
        in_specs=[
            pl.BlockSpec((NB, 64), lambda i: (i, 0)),
            pl.BlockSpec((64, 32), lambda i: (0, 0)),
            pl.BlockSpec((32,), lambda i: (0,)),
            pl.BlockSpec((32, 128), lambda i: (0, 0)),
            pl.BlockSpec((128,), lambda i: (0,)),
        ],
        out_specs=pl.BlockSpec((NB, 128), lambda i: (i, 0)),
        out_shape=jax.ShapeDtypeStruct((NP, 128), jnp.float32),
    )(h, Wlin, blin, w1p, b1p)


def _gat_layer(h, ss, ds, eas, rowptr, Wl, Wr, We, att, b, loop_attr):
    n = h.shape[0]
    xl = h @ Wl
    xr = h @ Wr
    m = xl[ss] + xr[ds] + eas @ We
    m = _leaky(m)
    logit = m @ att
    mx_r = jax.ops.segment_max(logit, ds, num_segments=n, indices_are_sorted=True)
    mx_r = jnp.where(jnp.isfinite(mx_r), mx_r, -1e30)
    ex = jnp.exp(logit - mx_r[ds])
    den_r = jax.ops.segment_sum(ex, ds, num_segments=n, indices_are_sorted=True)
    num_r = jax.ops.segment_sum(ex[:, None] * xl[ss], ds, num_segments=n,
                                indices_are_sorted=True)
    ls = _leaky(xl + xr + loop_attr @ We) @ att
    mxf = jnp.maximum(mx_r, ls)
    f = jnp.exp(mx_r - mxf)
    es = jnp.exp(ls - mxf)
    out = (num_r * f[:, None] + es[:, None] * xl) / (
        (den_r * f + es + 1e-16)[:, None])
    return out + b


def kernel(x, edge_index, edge_attr, Wl1, Wr1, We1, att1, b1, Wl2, Wr2, We2,
           att2, b2, Wl3, Wr3, We3, att3, b3, Wlin, blin, Wlin1, blin1):
    src, dst = edge_index[0], edge_index[1]
    perm = jnp.argsort(dst)
    ss = src[perm]
    ds = dst[perm]
    eas = edge_attr[perm]
    rowptr = jnp.searchsorted(ds, jnp.arange(N + 1, dtype=jnp.int32)).astype(jnp.int32)
    cnt = (rowptr[1:] - rowptr[:-1]).astype(jnp.float32)
    sum_attr = jax.ops.segment_sum(eas, ds, num_segments=N, indices_are_sorted=True)
    loop_attr = sum_attr / jnp.maximum(cnt, 1.0)[:, None]

    h = x
    h = jax.nn.relu(_gat_layer(h, ss, ds, eas, rowptr, Wl1, Wr1, We1, att1, b1, loop_attr))
    h = jax.nn.relu(_gat_layer(h, ss, ds, eas, rowptr, Wl2, Wr2, We2, att2, b2, loop_attr))
    h = jax.nn.relu(_gat_layer(h, ss, ds, eas, rowptr, Wl3, Wr3, We3, att3, b3, loop_attr))

    hp = jnp.zeros((NP, 64), jnp.float32).at[:N].set(h)
    out = _mlp_tail(hp, Wlin, blin, Wlin1, blin1)
    return out[:N, :4]


# speedup vs baseline: 3.0318x; 1.8543x over previous
import jax
import jax.numpy as jnp
from jax import lax
from jax.experimental import pallas as pl
from jax.experimental.pallas import tpu as pltpu
from jax.experimental.pallas import tpu_sc as plsc

N = 50000
E = 800000

NBK = 128
BLKS = 14
NW = 32
NP = NW * BLKS * NBK
W = 128
EPA = 800768
NB = 256

_NEG = -1e30


def _i16():
    return lax.iota(jnp.int32, 16)


def _splat(x, dtype=jnp.int32):
    return jnp.full((16,), x, dtype)


def _scalar_i(ref, pos):
    v = plsc.load_gather(ref, [jnp.full((16,), pos, jnp.int32)])
    return jnp.max(jnp.where(_i16() == 0, v, jnp.int32(-2147483648)))


def _gat_sc_body(ss, ds, rp, xl, xrls, ea, attf, bf, h_out,
                 idx_a, ds_st, rp_st, xl_st, ea_st, xrls_blk, xls_blk,
                 logit_st, acc, h_blk_a, h_blk_b, att_st, b_st,
                 sem1, semo_a, semo_b):
    wid = lax.axis_index("s") * 2 + lax.axis_index("c")
    iota = _i16()
    lane0 = iota == 0

    pltpu.sync_copy(attf, att_st)
    pltpu.sync_copy(bf, b_st)

    def do_block(blk, j, h_blk, semo):
        b0 = pl.multiple_of((wid * BLKS + blk) * NBK, NBK)
        pltpu.sync_copy(rp.at[pl.ds(b0, NBK + 8)], rp_st)
        pltpu.sync_copy(xrls.at[pl.ds(b0, NBK)], xrls_blk)
        pltpu.sync_copy(xl.at[pl.ds(b0, NBK)], xls_blk)

        def zi(j, _):
            plsc.store_scatter(acc, [j * 16 + iota], jnp.zeros((16,), jnp.float32))
            return 0
        lax.fori_loop(0, NBK * 66 // 16, zi, 0)

        def zm(j, _):
            plsc.store_scatter(acc, [(j * 16 + iota) * 66 + 65],
                               jnp.full((16,), _NEG, jnp.float32))
            return 0
        lax.fori_loop(0, NBK // 16, zm, 0)

        es0 = _scalar_i(rp_st, 0)
        es1 = _scalar_i(rp_st, NBK)
        es0a = jnp.bitwise_and(es0, jnp.int32(-8))
        nwin = (es1 - es0a + (W - 1)) >> 7

        def do_window(w, _):
            we0 = pl.multiple_of(es0a + w * W, 8)
            vs = jnp.maximum(we0, es0)
            ve = jnp.minimum(we0 + W, es1)

            @pl.when(ve > vs)
            def _():
                pltpu.sync_copy(ss.at[pl.ds(we0, W)], idx_a)
                pltpu.sync_copy(ds.at[pl.ds(we0, W)], ds_st)
                cp1 = pltpu.async_copy(xl.at[idx_a], xl_st, sem1)
                pltpu.sync_copy(ea.at[pl.ds(we0, W)], ea_st)
                cp1.wait()

                def pha(g, _):
                    ev = g * 16 + iota
                    dv = plsc.load_gather(ds_st, [ev])
                    dl = jnp.clip(dv - b0, 0, NBK - 1)
                    lg = jnp.zeros((16,), jnp.float32)
                    for f in range(64):
                        fs = _splat(f)
                        a = plsc.load_gather(xl_st, [ev, fs])
                        b = plsc.load_gather(ea_st, [ev, fs])
                        c = plsc.load_gather(xrls_blk, [dl, fs])
                        t = a + b + c
                        t = jnp.maximum(t, 0.2 * t)
                        lg = lg + t * att_st[pl.ds(f * 16, 16)]
                    plsc.store_scatter(logit_st, [ev], lg)
                    return 0
                lax.fori_loop(0, W // 16, pha, 0)

                d_first = _scalar_i(ds_st, vs - we0)
                d_last = _scalar_i(ds_st, ve - 1 - we0)

                def node(n, _):
                    rel = n - b0
                    rp0 = _scalar_i(rp_st, rel)
                    rp1 = _scalar_i(rp_st, rel + 1)
                    ep0 = jnp.maximum(rp0, vs)
                    ep1 = jnp.minimum(rp1, ve)

                    def mx_sweep(ep, m):
                        lgv = plsc.load_gather(logit_st, [_splat(ep - we0)])
                        return jnp.maximum(m, lgv)
                    m_w = lax.fori_loop(ep0, ep1, mx_sweep,
                                        jnp.full((16,), _NEG, jnp.float32))

                    def ex_sweep(ep, carry):
                        s, n0, n1, n2, n3 = carry
                        rr = ep - we0
                        lgv = plsc.load_gather(logit_st, [_splat(rr)])
                        wv = jnp.exp(lgv - m_w)
                        rs = _splat(rr)
                        x0 = plsc.load_gather(xl_st, [rs, iota])
                        x1 = plsc.load_gather(xl_st, [rs, 16 + iota])
                        x2 = plsc.load_gather(xl_st, [rs, 32 + iota])
                        x3 = plsc.load_gather(xl_st, [rs, 48 + iota])
                        return (s + wv, n0 + wv * x0, n1 + wv * x1,
                                n2 + wv * x2, n3 + wv * x3)
                    z = jnp.zeros((16,), jnp.float32)
                    s_w, n0, n1, n2, n3 = lax.fori_loop(
                        ep0, ep1, ex_sweep, (z, z, z, z, z))

                    base = _splat(rel * 66)
                    m_o = plsc.load_gather(acc, [base + 65])
                    s_o = plsc.load_gather(acc, [base + 64])
                    m_n = jnp.maximum(m_o, m_w)
                    f_o = jnp.exp(m_o - m_n)
                    f_w = jnp.exp(m_w - m_n)
                    plsc.store_scatter(acc, [base + 65], m_n, mask=lane0)
                    plsc.store_scatter(acc, [base + 64], s_o * f_o + s_w * f_w,
                                       mask=lane0)
                    for c, nc in enumerate((n0, n1, n2, n3)):
                        io = base + c * 16 + iota
                        no = plsc.load_gather(acc, [io])
                        plsc.store_scatter(acc, [io], no * f_o + nc * f_w)
                    return 0
                lax.fori_loop(d_first, d_last + 1, node, 0)
            return 0
        lax.fori_loop(0, nwin, do_window, 0)

        @pl.when(j > 0)
        def _():
            pltpu.make_async_copy(
                h_blk, h_out.at[pl.ds(0, NBK * 64)], semo).wait()

        def fin(g, _):
            nv = g * 16 + iota
            s_v = plsc.load_gather(acc, [nv * 66 + 64])
            m_v = plsc.load_gather(acc, [nv * 66 + 65])
            ls_v = plsc.load_gather(xrls_blk, [nv, _splat(64)])
            m_eff = jnp.where(s_v > 0.0, m_v, jnp.float32(_NEG))
            mxf = jnp.maximum(m_eff, ls_v)
            f_o = jnp.exp(m_eff - mxf)
            es = jnp.exp(ls_v - mxf)
            rden = 1.0 / (s_v * f_o + es + 1e-16)
            for f in range(64):
                fs = _splat(f)
                nm = plsc.load_gather(acc, [nv * 66 + f])
                xs = plsc.load_gather(xls_blk, [nv, fs])
                hv = (nm * f_o + es * xs) * rden + b_st[pl.ds(f * 16, 16)]
                hv = jnp.maximum(hv, 0.0)
                plsc.store_scatter(h_blk, [nv * 64 + fs], hv)
            return 0
        lax.fori_loop(0, NBK // 16, fin, 0)
        pltpu.async_copy(h_blk, h_out.at[pl.ds(b0 * 64, NBK * 64)], semo)

    def do_pair(j, _):
        do_block(2 * j, j, h_blk_a, semo_a)
        do_block(2 * j + 1, j, h_blk_b, semo_b)
        return 0
    lax.fori_loop(0, BLKS // 2, do_pair, 0)
    pltpu.make_async_copy(h_blk_a, h_out.at[pl.ds(0, NBK * 64)], semo_a).wait()
    pltpu.make_async_copy(h_blk_b, h_out.at[pl.ds(0, NBK * 64)], semo_b).wait()


def _gat_sc(ss, ds, rp, xl, xrls, ea, attf, bf):
    mesh = plsc.VectorSubcoreMesh(core_axis_name="c", subcore_axis_name="s")
    kfn = pl.kernel(
        _gat_sc_body,
        out_type=jax.ShapeDtypeStruct((NP * 64,), jnp.float32),
        mesh=mesh,
        compiler_params=pltpu.CompilerParams(needs_layout_passes=False),
        scratch_types=[
            pltpu.VMEM((W,), jnp.int32),
            pltpu.VMEM((W,), jnp.int32),
            pltpu.VMEM((NBK + 8,), jnp.int32),
            pltpu.VMEM((W, 128), jnp.float32),
            pltpu.VMEM((W, 64), jnp.float32),
            pltpu.VMEM((NBK, 128), jnp.float32),
            pltpu.VMEM((NBK, 128), jnp.float32),
            pltpu.VMEM((W,), jnp.float32),
            pltpu.VMEM((NBK * 66,), jnp.float32),
            pltpu.VMEM((NBK * 64,), jnp.float32),
            pltpu.VMEM((NBK * 64,), jnp.float32),
            pltpu.VMEM((1024,), jnp.float32),
            pltpu.VMEM((1024,), jnp.float32),
            pltpu.SemaphoreType.DMA,
            pltpu.SemaphoreType.DMA,
            pltpu.SemaphoreType.DMA,
        ],
    )
    return kfn(ss, ds, rp, xl, xrls, ea, attf, bf)


def _sum_sc_body(ds, rp, ea8, sum_out, ds_st, rp_st, ea_st, acc_a, acc_b,
                 semo_a, semo_b):
    wid = lax.axis_index("s") * 2 + lax.axis_index("c")
    iota = _i16()
    cmask = iota < 8

    def do_block(blk, j, acc, semo):
        b0 = pl.multiple_of((wid * BLKS + blk) * NBK, NBK)
        pltpu.sync_copy(rp.at[pl.ds(b0, NBK + 8)], rp_st)

        @pl.when(j > 0)
        def _():
            pltpu.make_async_copy(
                acc, sum_out.at[pl.ds(0, NBK * 8)], semo).wait()

        def zi(j, _):
            plsc.store_scatter(acc, [j * 16 + iota], jnp.zeros((16,), jnp.float32))
            return 0
        lax.fori_loop(0, NBK * 8 // 16, zi, 0)

        es0 = _scalar_i(rp_st, 0)
        es1 = _scalar_i(rp_st, NBK)
        es0a = jnp.bitwise_and(es0, jnp.int32(-8))
        nwin = (es1 - es0a + (W - 1)) >> 7

        def do_window(w, _):
            we0 = pl.multiple_of(es0a + w * W, 8)
            vs = jnp.maximum(we0, es0)
            ve = jnp.minimum(we0 + W, es1)

            @pl.when(ve > vs)
            def _():
                pltpu.sync_copy(ds.at[pl.ds(we0, W)], ds_st)
                pltpu.sync_copy(ea8.at[pl.ds(we0, W)], ea_st)
                d_first = _scalar_i(ds_st, vs - we0)
                d_last = _scalar_i(ds_st, ve - 1 - we0)

                def node(n, _):
                    rel = n - b0
                    rp0 = _scalar_i(rp_st, rel)
                    rp1 = _scalar_i(rp_st, rel + 1)
                    ep0 = jnp.maximum(rp0, vs)
                    ep1 = jnp.minimum(rp1, ve)

                    def sweep(ep, s8):
                        row = plsc.load_gather(ea_st, [_splat(ep - we0), iota],
                                               mask=cmask)
                        return s8 + jnp.where(cmask, row, 0.0)
                    s8 = lax.fori_loop(ep0, ep1, sweep, jnp.zeros((16,), jnp.float32))
                    io = _splat(rel * 8) + iota
                    old = plsc.load_gather(acc, [io], mask=cmask)
                    plsc.store_scatter(acc, [io], old + s8, mask=cmask)
                    return 0
                lax.fori_loop(d_first, d_last + 1, node, 0)
            return 0
        lax.fori_loop(0, nwin, do_window, 0)
        pltpu.async_copy(acc, sum_out.at[pl.ds(b0 * 8, NBK * 8)], semo)

    def do_pair(j, _):
        do_block(2 * j, j, acc_a, semo_a)
        do_block(2 * j + 1, j, acc_b, semo_b)
        return 0
    lax.fori_loop(0, BLKS // 2, do_pair, 0)
    pltpu.make_async_copy(acc_a, sum_out.at[pl.ds(0, NBK * 8)], semo_a).wait()
    pltpu.make_async_copy(acc_b, sum_out.at[pl.ds(0, NBK * 8)], semo_b).wait()


def _sum_sc(ds, rp, ea8):
    mesh = plsc.VectorSubcoreMesh(core_axis_name="c", subcore_axis_name="s")
    kfn = pl.kernel(
        _sum_sc_body,
        out_type=jax.ShapeDtypeStruct((NP * 8,), jnp.float32),
        mesh=mesh,
        compiler_params=pltpu.CompilerParams(needs_layout_passes=False),
        scratch_types=[
            pltpu.VMEM((W,), jnp.int32),
            pltpu.VMEM((NBK + 8,), jnp.int32),
            pltpu.VMEM((W, 8), jnp.float32),
            pltpu.VMEM((NBK * 8,), jnp.float32),
            pltpu.VMEM((NBK * 8,), jnp.float32),
            pltpu.SemaphoreType.DMA,
            pltpu.SemaphoreType.DMA,
        ],
    )
    return kfn(ds, rp, ea8)


def _proj_body(h_ref, wl_ref, wr_ref, we_ref, la_ref, att_ref,
               xl_ref, xrls_ref):
    h = h_ref[...]
    xl = h @ wl_ref[...]
    xr = h @ wr_ref[...]
    xl_ref[...] = jnp.concatenate([xl, jnp.zeros((NB, 64), jnp.float32)], axis=1)
    m = xl + xr + la_ref[...] @ we_ref[...]
    m = jnp.maximum(m, 0.2 * m)
    ls = m @ att_ref[...]
    xrls_ref[...] = jnp.concatenate([xr, jnp.broadcast_to(ls, (NB, 64))], axis=1)


def _proj(h, Wl, Wr, We8, la8, att):
    d = h.shape[1]
    grid = NP // NB
    return pl.pallas_call(
        _proj_body,
        grid=(grid,),
        in_specs=[
            pl.BlockSpec((NB, d), lambda i: (i, 0)),
            pl.BlockSpec((d, 64), lambda i: (0, 0)),
            pl.BlockSpec((d, 64), lambda i: (0, 0)),
            pl.BlockSpec((8, 64), lambda i: (0, 0)),
            pl.BlockSpec((NB, 8), lambda i: (i, 0)),
            pl.BlockSpec((64, 1), lambda i: (0, 0)),
        ],
        out_specs=[
            pl.BlockSpec((NB, 128), lambda i: (i, 0)),
            pl.BlockSpec((NB, 128), lambda i: (i, 0)),
        ],
        out_shape=[
            jax.ShapeDtypeStruct((NP, 128), jnp.float32),
            jax.ShapeDtypeStruct((NP, 128), jnp.float32),
        ],
    )(h, Wl, Wr, We8, la8, att)


def _eamm_body(ea_ref, we_ref, o_ref):
    o_ref[...] = ea_ref[...] @ we_ref[...]


def _eamm(ea8, We8):
    grid = EPA // 1024
    return pl.pallas_call(
        _eamm_body,
        grid=(grid,),
        in_specs=[
            pl.BlockSpec((1024, 8), lambda i: (i, 0)),
            pl.BlockSpec((8, 64), lambda i: (0, 0)),
        ],
        out_specs=pl.BlockSpec((1024, 64), lambda i: (i, 0)),
        out_shape=jax.ShapeDtypeStruct((EPA, 64), jnp.float32),
    )(ea8, We8)


def _mlp_body(h_ref, wlin_ref, blin_ref, wlin1_ref, blin1_ref, o_ref):
    t = jnp.maximum(h_ref[...] @ wlin_ref[...] + blin_ref[...], 0.0)
    o_ref[...] = t @ wlin1_ref[...] + blin1_ref[...]


def _mlp_tail(h, Wlin, blin, Wlin1, blin1):
    w1p = jnp.zeros((32, 128), jnp.float32).at[:, :4].set(Wlin1)
    b1p = jnp.zeros((128,), jnp.float32).at[:4].set(blin1)
    grid = NP // NB
    return pl.pallas_call(
        _mlp_body,
        grid=(grid,),
        in_specs=[
            pl.BlockSpec((NB, 64), lambda i: (i, 0)),
            pl.BlockSpec((64, 32), lambda i: (0, 0)),
            pl.BlockSpec((32,), lambda i: (0,)),
            pl.BlockSpec((32, 128), lambda i: (0, 0)),
            pl.BlockSpec((128,), lambda i: (0,)),
        ],
        out_specs=pl.BlockSpec((NB, 128), lambda i: (i, 0)),
        out_shape=jax.ShapeDtypeStruct((NP, 128), jnp.float32),
    )(h, Wlin, blin, w1p, b1p)


def _pad8(w):
    return jnp.zeros((8, 64), jnp.float32).at[:6].set(w)


def kernel(x, edge_index, edge_attr, Wl1, Wr1, We1, att1, b1, Wl2, Wr2, We2,
           att2, b2, Wl3, Wr3, We3, att3, b3, Wlin, blin, Wlin1, blin1):
    src, dst = edge_index[0], edge_index[1]
    perm = jnp.argsort(dst)
    ss = jnp.zeros((EPA,), jnp.int32).at[:E].set(src[perm])
    ds = jnp.full((EPA,), NP - 1, jnp.int32).at[:E].set(dst[perm])
    ea8 = jnp.zeros((EPA, 8), jnp.float32).at[:E, :6].set(edge_attr[perm])
    rowptr = jnp.full((NP + 8,), EPA, jnp.int32).at[:NP + 1].set(
        jnp.searchsorted(ds, jnp.arange(NP + 1, dtype=jnp.int32)).astype(jnp.int32))

    sum8 = _sum_sc(ds, rowptr, ea8).reshape(NP, 8)
    cnt = (rowptr[1:NP + 1] - rowptr[:NP]).astype(jnp.float32)
    la8 = sum8 / jnp.maximum(cnt, 1.0)[:, None]

    x8 = jnp.zeros((NP, 8), jnp.float32).at[:N, :6].set(x)

    h = x8
    for (Wlp, Wrp, We, att, b) in (
        (_pad8(Wl1), _pad8(Wr1), We1, att1, b1),
        (Wl2, Wr2, We2, att2, b2),
        (Wl3, Wr3, We3, att3, b3),
    ):
        We8 = _pad8(We)
        attc = att[:, None]
        attf = jnp.tile(att[:, None], (1, 16)).reshape(-1)
        bf = jnp.tile(b[:, None], (1, 16)).reshape(-1)
        xl128, xrls = _proj(h, Wlp, Wrp, We8, la8, attc)
        eaw = _eamm(ea8, We8)
        hflat = _gat_sc(ss, ds, rowptr, xl128, xrls, eaw, attf, bf)
        h = hflat.reshape(NP, 64)

    out = _mlp_tail(h, Wlin, blin, Wlin1, blin1)
    return out[:N, :4]


# speedup vs baseline: 3.1357x; 1.0343x over previous
import jax
import jax.numpy as jnp
from jax import lax
from jax.experimental import pallas as pl
from jax.experimental.pallas import tpu as pltpu
from jax.experimental.pallas import tpu_sc as plsc

N = 50000
E = 800000

NBK = 128
BLKS = 14
NW = 32
NP = NW * BLKS * NBK
W = 128
EPA = 800768
NB = 256

_NEG = -1e30


def _i16():
    return lax.iota(jnp.int32, 16)


def _splat(x, dtype=jnp.int32):
    return jnp.full((16,), x, dtype)


def _scalar_i(ref, pos):
    v = plsc.load_gather(ref, [jnp.full((16,), pos, jnp.int32)])
    return jnp.max(jnp.where(_i16() == 0, v, jnp.int32(-2147483648)))


def _gat_sc_body(ss, ds, rp, xl, xrls, ea, attf, bf, h_out,
                 idx_a, ds_st, rp_st, xl_st, ea_st, xrls_blk, xls_blk,
                 logit_st, acc, h_blk_a, h_blk_b, att_st, b_st,
                 sem1, sem2, sem3, semo_a, semo_b):
    wid = lax.axis_index("s") * 2 + lax.axis_index("c")
    iota = _i16()
    lane0 = iota == 0

    pltpu.sync_copy(attf, att_st)
    pltpu.sync_copy(bf, b_st)

    def do_block(blk, j, h_blk, semo):
        b0 = pl.multiple_of((wid * BLKS + blk) * NBK, NBK)
        cpr = pltpu.async_copy(rp.at[pl.ds(b0, NBK + 8)], rp_st, sem1)
        cpx = pltpu.async_copy(xrls.at[pl.ds(b0, NBK)], xrls_blk, sem2)
        cpl = pltpu.async_copy(xl.at[pl.ds(b0, NBK)], xls_blk, sem3)
        cpr.wait()
        cpx.wait()
        cpl.wait()

        def zi(j, _):
            plsc.store_scatter(acc, [j * 16 + iota], jnp.zeros((16,), jnp.float32))
            return 0
        lax.fori_loop(0, NBK * 66 // 16, zi, 0)

        def zm(j, _):
            plsc.store_scatter(acc, [(j * 16 + iota) * 66 + 65],
                               jnp.full((16,), _NEG, jnp.float32))
            return 0
        lax.fori_loop(0, NBK // 16, zm, 0)

        es0 = _scalar_i(rp_st, 0)
        es1 = _scalar_i(rp_st, NBK)
        es0a = jnp.bitwise_and(es0, jnp.int32(-8))
        nwin = (es1 - es0a + (W - 1)) >> 7

        def do_window(w, _):
            we0 = pl.multiple_of(es0a + w * W, 8)
            vs = jnp.maximum(we0, es0)
            ve = jnp.minimum(we0 + W, es1)

            @pl.when(ve > vs)
            def _():
                cpi = pltpu.async_copy(ss.at[pl.ds(we0, W)], idx_a, sem1)
                cpd = pltpu.async_copy(ds.at[pl.ds(we0, W)], ds_st, sem2)
                cpe = pltpu.async_copy(ea.at[pl.ds(we0, W)], ea_st, sem3)
                cpi.wait()
                cpg = pltpu.async_copy(xl.at[idx_a], xl_st, sem1)
                cpd.wait()
                cpe.wait()
                cpg.wait()

                def pha(g, _):
                    ev = g * 16 + iota
                    dv = plsc.load_gather(ds_st, [ev])
                    dl = jnp.clip(dv - b0, 0, NBK - 1)
                    lg = jnp.zeros((16,), jnp.float32)
                    for f in range(64):
                        fs = _splat(f)
                        a = plsc.load_gather(xl_st, [ev, fs])
                        b = plsc.load_gather(ea_st, [ev, fs])
                        c = plsc.load_gather(xrls_blk, [dl, fs])
                        t = a + b + c
                        t = jnp.maximum(t, 0.2 * t)
                        lg = lg + t * att_st[pl.ds(f * 16, 16)]
                    plsc.store_scatter(logit_st, [ev], lg)
                    return 0
                lax.fori_loop(0, W // 16, pha, 0)

                d_first = _scalar_i(ds_st, vs - we0)
                d_last = _scalar_i(ds_st, ve - 1 - we0)

                def node(n, _):
                    rel = n - b0
                    rp0 = _scalar_i(rp_st, rel)
                    rp1 = _scalar_i(rp_st, rel + 1)
                    ep0 = jnp.maximum(rp0, vs)
                    ep1 = jnp.minimum(rp1, ve)

                    def mx_sweep(ep, m):
                        lgv = plsc.load_gather(logit_st, [_splat(ep - we0)])
                        return jnp.maximum(m, lgv)
                    m_w = lax.fori_loop(ep0, ep1, mx_sweep,
                                        jnp.full((16,), _NEG, jnp.float32))

                    def ex_sweep(ep, carry):
                        s, n0, n1, n2, n3 = carry
                        rr = ep - we0
                        lgv = plsc.load_gather(logit_st, [_splat(rr)])
                        wv = jnp.exp(lgv - m_w)
                        rs = _splat(rr)
                        x0 = plsc.load_gather(xl_st, [rs, iota])
                        x1 = plsc.load_gather(xl_st, [rs, 16 + iota])
                        x2 = plsc.load_gather(xl_st, [rs, 32 + iota])
                        x3 = plsc.load_gather(xl_st, [rs, 48 + iota])
                        return (s + wv, n0 + wv * x0, n1 + wv * x1,
                                n2 + wv * x2, n3 + wv * x3)
                    z = jnp.zeros((16,), jnp.float32)
                    s_w, n0, n1, n2, n3 = lax.fori_loop(
                        ep0, ep1, ex_sweep, (z, z, z, z, z))

                    base = _splat(rel * 66)
                    m_o = plsc.load_gather(acc, [base + 65])
                    s_o = plsc.load_gather(acc, [base + 64])
                    m_n = jnp.maximum(m_o, m_w)
                    f_o = jnp.exp(m_o - m_n)
                    f_w = jnp.exp(m_w - m_n)
                    plsc.store_scatter(acc, [base + 65], m_n, mask=lane0)
                    plsc.store_scatter(acc, [base + 64], s_o * f_o + s_w * f_w,
                                       mask=lane0)
                    for c, nc in enumerate((n0, n1, n2, n3)):
                        io = base + c * 16 + iota
                        no = plsc.load_gather(acc, [io])
                        plsc.store_scatter(acc, [io], no * f_o + nc * f_w)
                    return 0
                lax.fori_loop(d_first, d_last + 1, node, 0)
            return 0
        lax.fori_loop(0, nwin, do_window, 0)

        @pl.when(j > 0)
        def _():
            pltpu.make_async_copy(
                h_blk, h_out.at[pl.ds(0, NBK * 64)], semo).wait()

        def fin(g, _):
            nv = g * 16 + iota
            s_v = plsc.load_gather(acc, [nv * 66 + 64])
            m_v = plsc.load_gather(acc, [nv * 66 + 65])
            ls_v = plsc.load_gather(xrls_blk, [nv, _splat(64)])
            m_eff = jnp.where(s_v > 0.0, m_v, jnp.float32(_NEG))
            mxf = jnp.maximum(m_eff, ls_v)
            f_o = jnp.exp(m_eff - mxf)
            es = jnp.exp(ls_v - mxf)
            rden = 1.0 / (s_v * f_o + es + 1e-16)
            for f in range(64):
                fs = _splat(f)
                nm = plsc.load_gather(acc, [nv * 66 + f])
                xs = plsc.load_gather(xls_blk, [nv, fs])
                hv = (nm * f_o + es * xs) * rden + b_st[pl.ds(f * 16, 16)]
                hv = jnp.maximum(hv, 0.0)
                plsc.store_scatter(h_blk, [nv * 64 + fs], hv)
            return 0
        lax.fori_loop(0, NBK // 16, fin, 0)
        pltpu.async_copy(h_blk, h_out.at[pl.ds(b0 * 64, NBK * 64)], semo)

    def do_pair(j, _):
        do_block(2 * j, j, h_blk_a, semo_a)
        do_block(2 * j + 1, j, h_blk_b, semo_b)
        return 0
    lax.fori_loop(0, BLKS // 2, do_pair, 0)
    pltpu.make_async_copy(h_blk_a, h_out.at[pl.ds(0, NBK * 64)], semo_a).wait()
    pltpu.make_async_copy(h_blk_b, h_out.at[pl.ds(0, NBK * 64)], semo_b).wait()


def _gat_sc(ss, ds, rp, xl, xrls, ea, attf, bf):
    mesh = plsc.VectorSubcoreMesh(core_axis_name="c", subcore_axis_name="s")
    kfn = pl.kernel(
        _gat_sc_body,
        out_type=jax.ShapeDtypeStruct((NP * 64,), jnp.float32),
        mesh=mesh,
        compiler_params=pltpu.CompilerParams(needs_layout_passes=False),
        scratch_types=[
            pltpu.VMEM((W,), jnp.int32),
            pltpu.VMEM((W,), jnp.int32),
            pltpu.VMEM((NBK + 8,), jnp.int32),
            pltpu.VMEM((W, 128), jnp.float32),
            pltpu.VMEM((W, 64), jnp.float32),
            pltpu.VMEM((NBK, 128), jnp.float32),
            pltpu.VMEM((NBK, 128), jnp.float32),
            pltpu.VMEM((W,), jnp.float32),
            pltpu.VMEM((NBK * 66,), jnp.float32),
            pltpu.VMEM((NBK * 64,), jnp.float32),
            pltpu.VMEM((NBK * 64,), jnp.float32),
            pltpu.VMEM((1024,), jnp.float32),
            pltpu.VMEM((1024,), jnp.float32),
            pltpu.SemaphoreType.DMA,
            pltpu.SemaphoreType.DMA,
            pltpu.SemaphoreType.DMA,
            pltpu.SemaphoreType.DMA,
            pltpu.SemaphoreType.DMA,
        ],
    )
    return kfn(ss, ds, rp, xl, xrls, ea, attf, bf)


def _sum_sc_body(ds, rp, ea8, sum_out, ds_st, rp_st, ea_st, acc_a, acc_b,
                 semo_a, semo_b):
    wid = lax.axis_index("s") * 2 + lax.axis_index("c")
    iota = _i16()
    cmask = iota < 8

    def do_block(blk, j, acc, semo):
        b0 = pl.multiple_of((wid * BLKS + blk) * NBK, NBK)
        pltpu.sync_copy(rp.at[pl.ds(b0, NBK + 8)], rp_st)

        @pl.when(j > 0)
        def _():
            pltpu.make_async_copy(
                acc, sum_out.at[pl.ds(0, NBK * 8)], semo).wait()

        def zi(j, _):
            plsc.store_scatter(acc, [j * 16 + iota], jnp.zeros((16,), jnp.float32))
            return 0
        lax.fori_loop(0, NBK * 8 // 16, zi, 0)

        es0 = _scalar_i(rp_st, 0)
        es1 = _scalar_i(rp_st, NBK)
        es0a = jnp.bitwise_and(es0, jnp.int32(-8))
        nwin = (es1 - es0a + (W - 1)) >> 7

        def do_window(w, _):
            we0 = pl.multiple_of(es0a + w * W, 8)
            vs = jnp.maximum(we0, es0)
            ve = jnp.minimum(we0 + W, es1)

            @pl.when(ve > vs)
            def _():
                pltpu.sync_copy(ds.at[pl.ds(we0, W)], ds_st)
                pltpu.sync_copy(ea8.at[pl.ds(we0, W)], ea_st)
                d_first = _scalar_i(ds_st, vs - we0)
                d_last = _scalar_i(ds_st, ve - 1 - we0)

                def node(n, _):
                    rel = n - b0
                    rp0 = _scalar_i(rp_st, rel)
                    rp1 = _scalar_i(rp_st, rel + 1)
                    ep0 = jnp.maximum(rp0, vs)
                    ep1 = jnp.minimum(rp1, ve)

                    def sweep(ep, s8):
                        row = plsc.load_gather(ea_st, [_splat(ep - we0), iota],
                                               mask=cmask)
                        return s8 + jnp.where(cmask, row, 0.0)
                    s8 = lax.fori_loop(ep0, ep1, sweep, jnp.zeros((16,), jnp.float32))
                    io = _splat(rel * 8) + iota
                    old = plsc.load_gather(acc, [io], mask=cmask)
                    plsc.store_scatter(acc, [io], old + s8, mask=cmask)
                    return 0
                lax.fori_loop(d_first, d_last + 1, node, 0)
            return 0
        lax.fori_loop(0, nwin, do_window, 0)
        pltpu.async_copy(acc, sum_out.at[pl.ds(b0 * 8, NBK * 8)], semo)

    def do_pair(j, _):
        do_block(2 * j, j, acc_a, semo_a)
        do_block(2 * j + 1, j, acc_b, semo_b)
        return 0
    lax.fori_loop(0, BLKS // 2, do_pair, 0)
    pltpu.make_async_copy(acc_a, sum_out.at[pl.ds(0, NBK * 8)], semo_a).wait()
    pltpu.make_async_copy(acc_b, sum_out.at[pl.ds(0, NBK * 8)], semo_b).wait()


def _sum_sc(ds, rp, ea8):
    mesh = plsc.VectorSubcoreMesh(core_axis_name="c", subcore_axis_name="s")
    kfn = pl.kernel(
        _sum_sc_body,
        out_type=jax.ShapeDtypeStruct((NP * 8,), jnp.float32),
        mesh=mesh,
        compiler_params=pltpu.CompilerParams(needs_layout_passes=False),
        scratch_types=[
            pltpu.VMEM((W,), jnp.int32),
            pltpu.VMEM((NBK + 8,), jnp.int32),
            pltpu.VMEM((W, 8), jnp.float32),
            pltpu.VMEM((NBK * 8,), jnp.float32),
            pltpu.VMEM((NBK * 8,), jnp.float32),
            pltpu.SemaphoreType.DMA,
            pltpu.SemaphoreType.DMA,
        ],
    )
    return kfn(ds, rp, ea8)


def _proj_body(h_ref, wl_ref, wr_ref, we_ref, la_ref, att_ref,
               xl_ref, xrls_ref):
    h = h_ref[...]
    xl = h @ wl_ref[...]
    xr = h @ wr_ref[...]
    xl_ref[...] = jnp.concatenate([xl, jnp.zeros((NB, 64), jnp.float32)], axis=1)
    m = xl + xr + la_ref[...] @ we_ref[...]
    m = jnp.maximum(m, 0.2 * m)
    ls = m @ att_ref[...]
    xrls_ref[...] = jnp.concatenate([xr, jnp.broadcast_to(ls, (NB, 64))], axis=1)


def _proj(h, Wl, Wr, We8, la8, att):
    d = h.shape[1]
    grid = NP // NB
    return pl.pallas_call(
        _proj_body,
        grid=(grid,),
        in_specs=[
            pl.BlockSpec((NB, d), lambda i: (i, 0)),
            pl.BlockSpec((d, 64), lambda i: (0, 0)),
            pl.BlockSpec((d, 64), lambda i: (0, 0)),
            pl.BlockSpec((8, 64), lambda i: (0, 0)),
            pl.BlockSpec((NB, 8), lambda i: (i, 0)),
            pl.BlockSpec((64, 1), lambda i: (0, 0)),
        ],
        out_specs=[
            pl.BlockSpec((NB, 128), lambda i: (i, 0)),
            pl.BlockSpec((NB, 128), lambda i: (i, 0)),
        ],
        out_shape=[
            jax.ShapeDtypeStruct((NP, 128), jnp.float32),
            jax.ShapeDtypeStruct((NP, 128), jnp.float32),
        ],
    )(h, Wl, Wr, We8, la8, att)


def _eamm_body(ea_ref, we_ref, o_ref):
    o_ref[...] = ea_ref[...] @ we_ref[...]


def _eamm(ea8, We8):
    grid = EPA // 1024
    return pl.pallas_call(
        _eamm_body,
        grid=(grid,),
        in_specs=[
            pl.BlockSpec((1024, 8), lambda i: (i, 0)),
            pl.BlockSpec((8, 64), lambda i: (0, 0)),
        ],
        out_specs=pl.BlockSpec((1024, 64), lambda i: (i, 0)),
        out_shape=jax.ShapeDtypeStruct((EPA, 64), jnp.float32),
    )(ea8, We8)


def _mlp_body(h_ref, wlin_ref, blin_ref, wlin1_ref, blin1_ref, o_ref):
    t = jnp.maximum(h_ref[...] @ wlin_ref[...] + blin_ref[...], 0.0)
    o_ref[...] = t @ wlin1_ref[...] + blin1_ref[...]


def _mlp_tail(h, Wlin, blin, Wlin1, blin1):
    w1p = jnp.zeros((32, 128), jnp.float32).at[:, :4].set(Wlin1)
    b1p = jnp.zeros((128,), jnp.float32).at[:4].set(blin1)
    grid = NP // NB
    return pl.pallas_call(
        _mlp_body,
        grid=(grid,),
        in_specs=[
            pl.BlockSpec((NB, 64), lambda i: (i, 0)),
            pl.BlockSpec((64, 32), lambda i: (0, 0)),
            pl.BlockSpec((32,), lambda i: (0,)),
            pl.BlockSpec((32, 128), lambda i: (0, 0)),
            pl.BlockSpec((128,), lambda i: (0,)),
        ],
        out_specs=pl.BlockSpec((NB, 128), lambda i: (i, 0)),
        out_shape=jax.ShapeDtypeStruct((NP, 128), jnp.float32),
    )(h, Wlin, blin, w1p, b1p)


def _pad8(w):
    return jnp.zeros((8, 64), jnp.float32).at[:6].set(w)


def kernel(x, edge_index, edge_attr, Wl1, Wr1, We1, att1, b1, Wl2, Wr2, We2,
           att2, b2, Wl3, Wr3, We3, att3, b3, Wlin, blin, Wlin1, blin1):
    src, dst = edge_index[0], edge_index[1]
    perm = jnp.argsort(dst)
    ss = jnp.zeros((EPA,), jnp.int32).at[:E].set(src[perm])
    ds = jnp.full((EPA,), NP - 1, jnp.int32).at[:E].set(dst[perm])
    ea8 = jnp.zeros((EPA, 8), jnp.float32).at[:E, :6].set(edge_attr[perm])
    rowptr = jnp.full((NP + 8,), EPA, jnp.int32).at[:NP + 1].set(
        jnp.searchsorted(ds, jnp.arange(NP + 1, dtype=jnp.int32)).astype(jnp.int32))

    sum8 = _sum_sc(ds, rowptr, ea8).reshape(NP, 8)
    cnt = (rowptr[1:NP + 1] - rowptr[:NP]).astype(jnp.float32)
    la8 = sum8 / jnp.maximum(cnt, 1.0)[:, None]

    x8 = jnp.zeros((NP, 8), jnp.float32).at[:N, :6].set(x)

    h = x8
    for (Wlp, Wrp, We, att, b) in (
        (_pad8(Wl1), _pad8(Wr1), We1, att1, b1),
        (Wl2, Wr2, We2, att2, b2),
        (Wl3, Wr3, We3, att3, b3),
    ):
        We8 = _pad8(We)
        attc = att[:, None]
        attf = jnp.tile(att[:, None], (1, 16)).reshape(-1)
        bf = jnp.tile(b[:, None], (1, 16)).reshape(-1)
        xl128, xrls = _proj(h, Wlp, Wrp, We8, la8, attc)
        eaw = _eamm(ea8, We8)
        hflat = _gat_sc(ss, ds, rowptr, xl128, xrls, eaw, attf, bf)
        h = hflat.reshape(NP, 64)

    out = _mlp_tail(h, Wlin, blin, Wlin1, blin1)
    return out[:N, :4]
